# baseline (speedup 1.0000x reference)
import functools

import numpy as np
import jax
import jax.numpy as jnp
from jax import lax
from jax.experimental import pallas as pl
from jax.experimental.pallas import tpu as pltpu
from jax.experimental.pallas import tpu_sc as plsc

B, L, V, E, O = 4096, 50, 100000, 128, 10000

NC = 2
NS = 16
NW = NC * NS
N_ROWS = B * L
ROWS_PER_W = N_ROWS // NW
CHUNK = 128
K_INFLIGHT = 2
BUF_ROWS = CHUNK * K_INFLIGHT
OUTER = ROWS_PER_W // BUF_ROWS

_sc_mesh = plsc.VectorSubcoreMesh(core_axis_name="c", subcore_axis_name="s")


@functools.partial(
    pl.kernel,
    mesh=_sc_mesh,
    out_type=jax.ShapeDtypeStruct((N_ROWS, E), jnp.float32),
    scratch_types=[
        pltpu.VMEM((ROWS_PER_W,), jnp.int32),
        pltpu.VMEM((2, BUF_ROWS, E), jnp.float32),
        pltpu.SemaphoreType.DMA,
        pltpu.SemaphoreType.DMA,
    ],
)
def _sc_gather(idx_hbm, table_hbm, out_hbm, idx_v, rows_v, gsem, wsem):
    wid = lax.axis_index("s") * NC + lax.axis_index("c")
    base = wid * ROWS_PER_W
    pltpu.sync_copy(idx_hbm.at[pl.ds(base, ROWS_PER_W)], idx_v)
    for outer in range(OUTER):
        p = outer % 2
        o0 = outer * BUF_ROWS
        if outer >= 2:
            prev0 = (outer - 2) * BUF_ROWS
            pltpu.make_async_copy(
                rows_v.at[p], out_hbm.at[pl.ds(base + prev0, BUF_ROWS)], wsem
            ).wait()
        gathers = []
        for j in range(K_INFLIGHT):
            gathers.append(
                pltpu.async_copy(
                    table_hbm.at[idx_v.at[pl.ds(o0 + j * CHUNK, CHUNK)]],
                    rows_v.at[p, pl.ds(j * CHUNK, CHUNK)],
                    gsem,
                )
            )
        for g in gathers:
            g.wait()
        pltpu.async_copy(
            rows_v.at[p], out_hbm.at[pl.ds(base + o0, BUF_ROWS)], wsem
        )
    for tail in (OUTER - 2, OUTER - 1):
        pltpu.make_async_copy(
            rows_v.at[tail % 2],
            out_hbm.at[pl.ds(base + tail * BUF_ROWS, BUF_ROWS)],
            wsem,
        ).wait()


BBLK = 256
GAMMA = np.float32(np.float32(0.9) * np.float32(L - 1) - np.float32(44.0))


def _tc_body(g_ref, wt_ref, eb_ref, b_ref, out_ref, ob, wsem):
    i = pl.program_id(0)
    nsteps = pl.num_programs(0)
    par = i % 3

    @pl.when(i >= 3)
    def _drain():
        pltpu.make_async_copy(
            ob.at[par], out_ref.at[pl.ds((i - 3) * BBLK, BBLK)], wsem
        ).wait()

    neg_inf = jnp.float32(-jnp.inf)
    top = [jnp.full((BBLK, E), neg_inf, jnp.float32) for _ in range(6)]
    for l in range(L):
        x = g_ref[l]
        for k in range(6):
            hi = jnp.maximum(top[k], x)
            x = jnp.minimum(top[k], x)
            top[k] = hi
    qs = top[5] + GAMMA * (top[4] - top[5])
    acc = jnp.zeros((BBLK, E), jnp.float32)
    for l in range(L):
        x = g_ref[l]
        acc = acc + jnp.where(x >= qs, x, 0.0)
    s = acc + eb_ref[...]
    ob[par] = (
        jnp.dot(s, wt_ref[...], preferred_element_type=jnp.float32)
        + b_ref[...]
    )
    cp = pltpu.make_async_copy(
        ob.at[par], out_ref.at[pl.ds(i * BBLK, BBLK)], wsem
    )
    cp.start()

    @pl.when(i == nsteps - 1)
    def _tail():
        for back in (2, 1, 0):
            pltpu.make_async_copy(
                ob.at[(i - back) % 3],
                out_ref.at[pl.ds((i - back) * BBLK, BBLK)],
                wsem,
            ).wait()


_tc_call = pl.pallas_call(
    _tc_body,
    grid=(B // BBLK,),
    in_specs=[
        pl.BlockSpec((L, BBLK, E), lambda i: (0, i, 0)),
        pl.BlockSpec((E, O), lambda i: (0, 0)),
        pl.BlockSpec((1, E), lambda i: (0, 0)),
        pl.BlockSpec((1, O), lambda i: (0, 0)),
    ],
    out_specs=pl.BlockSpec(memory_space=pltpu.MemorySpace.HBM),
    out_shape=jax.ShapeDtypeStruct((B, O), jnp.float32),
    scratch_shapes=[
        pltpu.VMEM((3, BBLK, O), jnp.float32),
        pltpu.SemaphoreType.DMA,
    ],
)


def kernel(tokens, emb_table, emb_bias, W, b):
    idx = tokens.astype(jnp.int32).T.reshape(-1)
    gathered = _sc_gather(idx, emb_table)
    g3 = gathered.reshape(L, B, E)
    return _tc_call(g3, W.T, emb_bias.reshape(1, E), b.reshape(1, O))
